# Initial kernel scaffold; baseline (speedup 1.0000x reference)
import functools

import jax
import jax.numpy as jnp
from jax import lax
from jax.experimental import pallas as pl
from jax.experimental.pallas import tpu as pltpu
from jax.experimental.pallas import tpu_sc as plsc

NUM_VOCAB = 1000
EMBED = 128
RESOLUTION = 32
B = 1024
L = 200
N = B * L

SMALL_ROWS = (RESOLUTION + 1) + 3 * (2 * RESOLUTION)

LANES = 16
NC = 2
NS = 16
NW = NC * NS

ROWS_PER_W = N // NW
CHUNK = 128
NCHUNK = ROWS_PER_W // CHUNK
TOT_CHUNKS = N // CHUNK

_mesh = plsc.VectorSubcoreMesh(core_axis_name="c", subcore_axis_name="s")


@functools.partial(
    pl.kernel,
    out_type=jax.ShapeDtypeStruct((N, EMBED), jnp.float32),
    mesh=_mesh,
    scratch_types=[
        pltpu.VMEM((SMALL_ROWS * EMBED,), jnp.float32),
        pltpu.VMEM((CHUNK,), jnp.int32),
        pltpu.VMEM((CHUNK, EMBED), jnp.float32),
        pltpu.SMEM((4, CHUNK), jnp.int32),
        pltpu.SemaphoreType.DMA,
    ],
)
def _embed_sum_kernel(vidx_hbm, sidx_hbm, vtab_hbm, stab_hbm, out_hbm,
                      stab_v, vidx_v, rows_v, sidx_s, sem):
    wid = lax.axis_index("s") * NC + lax.axis_index("c")
    pltpu.sync_copy(stab_hbm, stab_v)

    def chunk_body(k, carry):
        gk = wid * NCHUNK + k
        base = gk * CHUNK
        pltpu.sync_copy(vidx_hbm.at[pl.ds(base, CHUNK)], vidx_v)
        pltpu.sync_copy(sidx_hbm.at[gk], sidx_s)
        pltpu.async_copy(vtab_hbm.at[vidx_v], rows_v, sem).wait()

        def row_body(r, carry2):
            o0 = sidx_s[0, r] * EMBED
            o1 = sidx_s[1, r] * EMBED
            o2 = sidx_s[2, r] * EMBED
            o3 = sidx_s[3, r] * EMBED
            for j in range(EMBED // LANES):
                acc = rows_v[r, pl.ds(j * LANES, LANES)]
                acc = acc + stab_v[pl.ds(o0 + j * LANES, LANES)]
                acc = acc + stab_v[pl.ds(o1 + j * LANES, LANES)]
                acc = acc + stab_v[pl.ds(o2 + j * LANES, LANES)]
                acc = acc + stab_v[pl.ds(o3 + j * LANES, LANES)]
                rows_v[r, pl.ds(j * LANES, LANES)] = acc
            return carry2

        lax.fori_loop(0, CHUNK, row_body, 0)
        pltpu.sync_copy(rows_v, out_hbm.at[pl.ds(base, CHUNK)])
        return carry

    lax.fori_loop(0, NCHUNK, chunk_body, 0)


def kernel(value, depth, position, value_table, depth_table, pos_table):
    vt = value_table.at[0].set(0.0)
    stab = jnp.concatenate(
        [depth_table, pos_table[0], pos_table[1], pos_table[2]], axis=0
    ).reshape(-1)

    vidx = value.reshape(N)
    off_p0 = RESOLUTION + 1
    off_p1 = off_p0 + 2 * RESOLUTION
    off_p2 = off_p1 + 2 * RESOLUTION
    sidx = jnp.stack(
        [
            depth.reshape(N),
            position[..., 0].reshape(N) + off_p0,
            position[..., 1].reshape(N) + off_p1,
            position[..., 2].reshape(N) + off_p2,
        ],
        axis=0,
    )
    sidx = sidx.reshape(4, TOT_CHUNKS, CHUNK).transpose(1, 0, 2)

    out = _embed_sum_kernel(vidx, sidx, vt, stab)
    return out.reshape(B, L, EMBED)


# speedup vs baseline: 5.8176x; 5.8176x over previous
import functools

import jax
import jax.numpy as jnp
from jax import lax
from jax.experimental import pallas as pl
from jax.experimental.pallas import tpu as pltpu
from jax.experimental.pallas import tpu_sc as plsc

NUM_VOCAB = 1000
EMBED = 128
RESOLUTION = 32
B = 1024
L = 200
N = B * L

SMALL_ROWS = (RESOLUTION + 1) + 3 * (2 * RESOLUTION)

LANES = 16
NC = 2
NS = 16
NW = NC * NS

ROWS_PER_W = N // NW
CHUNK = 128
NCHUNK = ROWS_PER_W // CHUNK
TOT_CHUNKS = N // CHUNK

_mesh = plsc.VectorSubcoreMesh(core_axis_name="c", subcore_axis_name="s")


@functools.partial(
    pl.kernel,
    out_type=jax.ShapeDtypeStruct((N, EMBED), jnp.float32),
    mesh=_mesh,
    scratch_types=[
        pltpu.VMEM((SMALL_ROWS * EMBED,), jnp.float32),
        pltpu.VMEM((CHUNK,), jnp.int32),
        pltpu.VMEM((CHUNK, EMBED), jnp.float32),
        pltpu.VMEM((4, CHUNK), jnp.int32),
        pltpu.SemaphoreType.DMA,
    ],
)
def _embed_sum_kernel(vidx_hbm, sidx_hbm, vtab_hbm, stab_hbm, out_hbm,
                      stab_v, vidx_v, rows_v, sidx_v, sem):
    wid = lax.axis_index("s") * NC + lax.axis_index("c")
    pltpu.sync_copy(stab_hbm, stab_v)

    def chunk_body(k, carry):
        gk = wid * NCHUNK + k
        base = gk * CHUNK
        pltpu.sync_copy(vidx_hbm.at[pl.ds(base, CHUNK)], vidx_v)
        pltpu.sync_copy(sidx_hbm.at[gk], sidx_v)
        pltpu.async_copy(vtab_hbm.at[vidx_v], rows_v, sem).wait()

        def group_body(g, carry2):
            rbase = g * LANES
            s0 = sidx_v[0, pl.ds(rbase, LANES)] * EMBED
            s1 = sidx_v[1, pl.ds(rbase, LANES)] * EMBED
            s2 = sidx_v[2, pl.ds(rbase, LANES)] * EMBED
            s3 = sidx_v[3, pl.ds(rbase, LANES)] * EMBED
            for lane in range(LANES):
                r = rbase + lane
                o0, o1, o2, o3 = s0[lane], s1[lane], s2[lane], s3[lane]
                for j in range(EMBED // LANES):
                    acc = rows_v[r, pl.ds(j * LANES, LANES)]
                    acc = acc + stab_v[pl.ds(o0 + j * LANES, LANES)]
                    acc = acc + stab_v[pl.ds(o1 + j * LANES, LANES)]
                    acc = acc + stab_v[pl.ds(o2 + j * LANES, LANES)]
                    acc = acc + stab_v[pl.ds(o3 + j * LANES, LANES)]
                    rows_v[r, pl.ds(j * LANES, LANES)] = acc
            return carry2

        lax.fori_loop(0, CHUNK // LANES, group_body, 0)
        pltpu.sync_copy(rows_v, out_hbm.at[pl.ds(base, CHUNK)])
        return carry

    lax.fori_loop(0, NCHUNK, chunk_body, 0)


def kernel(value, depth, position, value_table, depth_table, pos_table):
    vt = value_table.at[0].set(0.0)
    stab = jnp.concatenate(
        [depth_table, pos_table[0], pos_table[1], pos_table[2]], axis=0
    ).reshape(-1)

    vidx = value.reshape(N)
    off_p0 = RESOLUTION + 1
    off_p1 = off_p0 + 2 * RESOLUTION
    off_p2 = off_p1 + 2 * RESOLUTION
    sidx = jnp.stack(
        [
            depth.reshape(N),
            position[..., 0].reshape(N) + off_p0,
            position[..., 1].reshape(N) + off_p1,
            position[..., 2].reshape(N) + off_p2,
        ],
        axis=0,
    )
    sidx = sidx.reshape(4, TOT_CHUNKS, CHUNK).transpose(1, 0, 2)

    out = _embed_sum_kernel(vidx, sidx, vt, stab)
    return out.reshape(B, L, EMBED)


# speedup vs baseline: 9.7515x; 1.6762x over previous
import functools

import jax
import jax.numpy as jnp
from jax import lax
from jax.experimental import pallas as pl
from jax.experimental.pallas import tpu as pltpu
from jax.experimental.pallas import tpu_sc as plsc

NUM_VOCAB = 1000
EMBED = 128
RESOLUTION = 32
B = 1024
L = 200
N = B * L

ND = RESOLUTION + 1
NP = 2 * RESOLUTION

LANES = 16
NC = 2
NS = 16
NW = NC * NS

ROWS_PER_W = N // NW
CHUNK = 128
NCHUNK = ROWS_PER_W // CHUNK

_mesh = plsc.VectorSubcoreMesh(core_axis_name="c", subcore_axis_name="s")


def _build_tables_body(dt_ref, p0_ref, p1_ref, p2_ref, dp2_ref, p01_ref):
    dp2 = dt_ref[...][:, None, :] + p2_ref[...][None, :, :]
    dp2_ref[...] = dp2.reshape(ND * NP, EMBED)
    p01 = p0_ref[...][:, None, :] + p1_ref[...][None, :, :]
    p01_ref[...] = p01.reshape(NP * NP, EMBED)


def _build_tables(dt, p0, p1, p2):
    return pl.pallas_call(
        _build_tables_body,
        out_shape=(
            jax.ShapeDtypeStruct((ND * NP, EMBED), jnp.float32),
            jax.ShapeDtypeStruct((NP * NP, EMBED), jnp.float32),
        ),
    )(dt, p0, p1, p2)


@functools.partial(
    pl.kernel,
    out_type=jax.ShapeDtypeStruct((N, EMBED), jnp.float32),
    mesh=_mesh,
    scratch_types=[
        pltpu.VMEM((4, CHUNK), jnp.int32),
        pltpu.VMEM((CHUNK,), jnp.int32),
        pltpu.VMEM((CHUNK,), jnp.int32),
        pltpu.VMEM((CHUNK,), jnp.int32),
        pltpu.VMEM((CHUNK, EMBED), jnp.float32),
        pltpu.VMEM((CHUNK, EMBED), jnp.float32),
        pltpu.VMEM((CHUNK, EMBED), jnp.float32),
        pltpu.SemaphoreType.DMA,
    ],
)
def _embed_sum_kernel(vidx_hbm, sidx_hbm, vtab_hbm, dp2_hbm, p01_hbm, out_hbm,
                      sidx_v, vidx_v, idp2_v, ip01_v, bufa, bufb, bufc, sem):
    wid = lax.axis_index("s") * NC + lax.axis_index("c")

    def chunk_body(k, carry):
        gk = wid * NCHUNK + k
        base = gk * CHUNK
        pltpu.sync_copy(vidx_hbm.at[pl.ds(base, CHUNK)], vidx_v)
        pltpu.sync_copy(sidx_hbm.at[gk], sidx_v)

        for j in range(CHUNK // LANES):
            sl = pl.ds(j * LANES, LANES)
            idp2_v[sl] = sidx_v[0, sl] * NP + sidx_v[3, sl]
            ip01_v[sl] = sidx_v[1, sl] * NP + sidx_v[2, sl]

        cpa = pltpu.async_copy(vtab_hbm.at[vidx_v], bufa, sem)
        cpb = pltpu.async_copy(dp2_hbm.at[idp2_v], bufb, sem)
        cpc = pltpu.async_copy(p01_hbm.at[ip01_v], bufc, sem)
        cpa.wait()
        cpb.wait()
        cpc.wait()

        def row_body(r, carry2):
            for j in range(EMBED // LANES):
                sl = pl.ds(j * LANES, LANES)
                bufa[r, sl] = (bufa[r, sl] + bufb[r, sl]) + bufc[r, sl]
            return carry2

        lax.fori_loop(0, CHUNK, row_body, 0)
        pltpu.sync_copy(bufa, out_hbm.at[pl.ds(base, CHUNK)])
        return carry

    lax.fori_loop(0, NCHUNK, chunk_body, 0)


def kernel(value, depth, position, value_table, depth_table, pos_table):
    vt = value_table.at[0].set(0.0)
    dp2, p01 = _build_tables(
        depth_table, pos_table[0], pos_table[1], pos_table[2]
    )

    vidx = value.reshape(N)
    sidx = jnp.stack(
        [
            depth.reshape(N),
            position[..., 0].reshape(N),
            position[..., 1].reshape(N),
            position[..., 2].reshape(N),
        ],
        axis=0,
    )
    sidx = sidx.reshape(4, N // CHUNK, CHUNK).transpose(1, 0, 2)

    out = _embed_sum_kernel(vidx, sidx, vt, dp2, p01)
    return out.reshape(B, L, EMBED)
